# baseline (speedup 1.0000x reference)
import jax
import jax.numpy as jnp
from jax.experimental import pallas as pl
from jax.experimental.pallas import tpu as pltpu

_NUM_NODES = 1000000
_EMBED_DIM = 64
_NCHUNK = 8
_ROWS = _NUM_NODES // _NCHUNK


def _copy_body(w_ref, o_ref, sems):
    for i in range(_NCHUNK):
        pltpu.make_async_copy(
            w_ref.at[pl.ds(i * _ROWS, _ROWS)],
            o_ref.at[pl.ds(i * _ROWS, _ROWS)],
            sems.at[i],
        ).start()
    for i in range(_NCHUNK):
        pltpu.make_async_copy(
            w_ref.at[pl.ds(i * _ROWS, _ROWS)],
            o_ref.at[pl.ds(i * _ROWS, _ROWS)],
            sems.at[i],
        ).wait()


def kernel(weight):
    return pl.pallas_call(
        _copy_body,
        out_shape=jax.ShapeDtypeStruct((_NUM_NODES, _EMBED_DIM), jnp.float32),
        in_specs=[pl.BlockSpec(memory_space=pl.ANY)],
        out_specs=pl.BlockSpec(memory_space=pl.ANY),
        scratch_shapes=[pltpu.SemaphoreType.DMA((_NCHUNK,))],
    )(weight)


# speedup vs baseline: 16.1339x; 16.1339x over previous
import jax
import jax.numpy as jnp
from jax.experimental import pallas as pl
from jax.experimental.pallas import tpu as pltpu

_NUM_NODES = 1000000
_EMBED_DIM = 64
_BLOCK_ROWS = 25000
_GRID = _NUM_NODES // _BLOCK_ROWS


def _copy_body(w_ref, o_ref):
    o_ref[...] = w_ref[...]


def kernel(weight):
    return pl.pallas_call(
        _copy_body,
        out_shape=jax.ShapeDtypeStruct((_NUM_NODES, _EMBED_DIM), jnp.float32),
        grid=(_GRID,),
        in_specs=[pl.BlockSpec((_BLOCK_ROWS, _EMBED_DIM), lambda i: (i, 0))],
        out_specs=pl.BlockSpec((_BLOCK_ROWS, _EMBED_DIM), lambda i: (i, 0)),
    )(weight)
